# baseline (speedup 1.0000x reference)
import functools
import jax
import jax.numpy as jnp
from jax import lax
from jax.experimental import pallas as pl
from jax.experimental.pallas import tpu as pltpu
from jax.experimental.pallas import tpu_sc as plsc

_B = 16384
_D = 64
_H = 128
_L = 16

_info = plsc.get_sparse_core_info()
_NC, _NS = _info.num_cores, _info.num_subcores
_NW = _NC * _NS
_BPW = _B // _NW
_NPH = 4
_PH = _BPW // _NPH
_NSTR = 8
_CH = _PH // _NSTR

_sc_mesh = plsc.VectorSubcoreMesh(core_axis_name="c", subcore_axis_name="s")


@functools.partial(
    pl.kernel,
    out_type=(
        jax.ShapeDtypeStruct((_B, _H), jnp.float32),
        jax.ShapeDtypeStruct((_B, _H), jnp.float32),
    ),
    mesh=_sc_mesh,
    scratch_types=[
        pltpu.VMEM((_BPW,), jnp.int32),
        pltpu.VMEM((_BPW,), jnp.int32),
        pltpu.VMEM((_BPW,), jnp.int32),
        pltpu.VMEM((_BPW,), jnp.int32),
        pltpu.VMEM((2, _PH, _H), jnp.float32),
        pltpu.VMEM((2, _PH, _H), jnp.float32),
        pltpu.SemaphoreType.DMA,
        pltpu.SemaphoreType.DMA,
        pltpu.SemaphoreType.DMA,
    ],
    compiler_params=pltpu.CompilerParams(use_tc_tiling_on_sc=False),
)
def _sc_gather(user_hbm, art_hbm, utab2_hbm, atab2_hbm, xu_hbm, xa_hbm,
               idx_u, idx_a, pid_u, pid_a, slab_u, slab_a,
               sem_u, sem_a, sem_o):
    wid = lax.axis_index("s") * _NC + lax.axis_index("c")
    base = wid * _BPW
    pltpu.sync_copy(user_hbm.at[pl.ds(base, _BPW)], idx_u)
    pltpu.sync_copy(art_hbm.at[pl.ds(base, _BPW)], idx_a)

    def pairify(j, carry):
        s = pl.ds(j * _L, _L)
        pid_u[s] = lax.shift_right_logical(idx_u[s], 1)
        pid_a[s] = lax.shift_right_logical(idx_a[s], 1)
        return carry

    lax.fori_loop(0, _BPW // _L, pairify, 0)

    def phase(h, carry):
        off = h * _PH
        p = lax.rem(h, 2)

        @pl.when(h >= 2)
        def _():
            pltpu.make_async_copy(
                slab_u.at[p], xu_hbm.at[pl.ds(base, _PH)], sem_o).wait()
            pltpu.make_async_copy(
                slab_a.at[p], xa_hbm.at[pl.ds(base, _PH)], sem_o).wait()

        def issue(j, c):
            o = j * _CH
            pltpu.async_copy(
                utab2_hbm.at[pid_u.at[pl.ds(off + o, _CH)]],
                slab_u.at[p, pl.ds(o, _CH)], sem_u)
            pltpu.async_copy(
                atab2_hbm.at[pid_a.at[pl.ds(off + o, _CH)]],
                slab_a.at[p, pl.ds(o, _CH)], sem_a)
            return c

        lax.fori_loop(0, _NSTR, issue, 0)

        def drain(j, c):
            pltpu.make_async_copy(
                utab2_hbm.at[pl.ds(0, _CH)], slab_u.at[0, pl.ds(0, _CH)],
                sem_u).wait()
            pltpu.make_async_copy(
                atab2_hbm.at[pl.ds(0, _CH)], slab_a.at[0, pl.ds(0, _CH)],
                sem_a).wait()
            return c

        lax.fori_loop(0, _NSTR, drain, 0)

        pltpu.async_copy(
            slab_u.at[p], xu_hbm.at[pl.ds(base + off, _PH)], sem_o)
        pltpu.async_copy(
            slab_a.at[p], xa_hbm.at[pl.ds(base + off, _PH)], sem_o)
        return carry

    lax.fori_loop(0, _NPH, phase, 0)

    def drain_out(j, carry):
        pltpu.make_async_copy(
            slab_u.at[0], xu_hbm.at[pl.ds(base, _PH)], sem_o).wait()
        pltpu.make_async_copy(
            slab_a.at[0], xa_hbm.at[pl.ds(base, _PH)], sem_o).wait()
        return carry

    lax.fori_loop(0, 2, drain_out, 0)


_BLK = 2048


def _mlp_body(xu_ref, xa_ref, up_ref, ap_ref, w1a_ref, w1b_ref, b1_ref,
              w2_ref, b2_ref, out_ref):
    up = (up_ref[...] & 1) == 1
    ap = (ap_ref[...] & 1) == 1
    ue = jnp.where(up, xu_ref[:, _D:], xu_ref[:, :_D])
    ae = jnp.where(ap, xa_ref[:, _D:], xa_ref[:, :_D])
    h = jnp.dot(ue, w1a_ref[...], preferred_element_type=jnp.float32)
    h += jnp.dot(ae, w1b_ref[...], preferred_element_type=jnp.float32)
    h = jnp.maximum(h + b1_ref[...], 0.0)
    o = jnp.dot(h, w2_ref[...], preferred_element_type=jnp.float32)
    out_ref[...] = jax.nn.sigmoid(o + b2_ref[...])


_mlp = pl.pallas_call(
    _mlp_body,
    grid=(_B // _BLK,),
    in_specs=[
        pl.BlockSpec((_BLK, _H), lambda i: (i, 0)),
        pl.BlockSpec((_BLK, _H), lambda i: (i, 0)),
        pl.BlockSpec((_BLK, 1), lambda i: (i, 0)),
        pl.BlockSpec((_BLK, 1), lambda i: (i, 0)),
        pl.BlockSpec((_D, _H), lambda i: (0, 0)),
        pl.BlockSpec((_D, _H), lambda i: (0, 0)),
        pl.BlockSpec((1, _H), lambda i: (0, 0)),
        pl.BlockSpec((_H, 1), lambda i: (0, 0)),
        pl.BlockSpec((1, 1), lambda i: (0, 0)),
    ],
    out_specs=pl.BlockSpec((_BLK, 1), lambda i: (i, 0)),
    out_shape=jax.ShapeDtypeStruct((_B, 1), jnp.float32),
)


@jax.jit
def kernel(user, artwork, user_table, artwork_table, W1, b1, W2, b2):
    ut2 = user_table.reshape(-1, _H)
    at2 = artwork_table.reshape(-1, _H)
    xu, xa = _sc_gather(user, artwork, ut2, at2)
    return _mlp(xu, xa, user.reshape(_B, 1), artwork.reshape(_B, 1),
                W1[:, :_D].T, W1[:, _D:].T, b1.reshape(1, _H), W2.T,
                b2.reshape(1, 1))
